# Initial kernel scaffold; baseline (speedup 1.0000x reference)
import jax
import jax.numpy as jnp
from jax.experimental import pallas as pl


def kernel(query, key, value, Wq, bq, Wk, bk, Wv, bv, Wo, bo, splat_centers, splat_scales):
    raise NotImplementedError("write your pallas kernel here")


# speedup vs baseline: 1.7292x; 1.7292x over previous
import functools

import jax
import jax.numpy as jnp
from jax.experimental import pallas as pl
from jax.experimental.pallas import tpu as pltpu

_BS = 512


def _phase1_body(q_ref, v_ref, wqt_ref, bq_ref, ct_ref, sc_ref,
                 phi_ref, cs_ref, p_ref, wq_scr):
    i = pl.program_id(1)

    @pl.when(i == 0)
    def _():
        wq_scr[...] = wqt_ref[...].astype(jnp.bfloat16)

    q = jnp.dot(q_ref[0].astype(jnp.bfloat16), wq_scr[...],
                preferred_element_type=jnp.float32) + bq_ref[...]
    t2 = jnp.sum(q * q, axis=-1, keepdims=True)
    ct = ct_ref[...]
    c2 = jnp.sum(ct * ct, axis=0, keepdims=True)
    qc = jnp.dot(q.astype(jnp.bfloat16), ct.astype(jnp.bfloat16),
                 preferred_element_type=jnp.float32)
    d2 = jnp.maximum(t2 + c2 - 2.0 * qc, 0.0)
    inv = 0.5 / (sc_ref[...] * sc_ref[...])
    phi = jnp.exp(-d2 * inv)
    phi_ref[0] = phi
    ps = jnp.sum(phi, axis=0, keepdims=True)
    pv = jax.lax.dot_general(phi.astype(jnp.bfloat16),
                             v_ref[0].astype(jnp.bfloat16),
                             (((0,), (0,)), ((), ())),
                             preferred_element_type=jnp.float32)

    @pl.when(i == 0)
    def _():
        cs_ref[0] = ps
        p_ref[0] = pv

    @pl.when(i > 0)
    def _():
        cs_ref[0] += ps
        p_ref[0] += pv


def _phase2_body(p_ref, cs_ref, wvt_ref, bv_ref, wot_ref, m2_ref):
    pv = jnp.dot(p_ref[0].astype(jnp.bfloat16),
                 wvt_ref[...].astype(jnp.bfloat16),
                 preferred_element_type=jnp.float32)
    cs_col = jnp.transpose(cs_ref[0])
    pv = pv + cs_col * bv_ref[...]
    m2_ref[0] = jnp.dot(pv.astype(jnp.bfloat16),
                        wot_ref[...].astype(jnp.bfloat16),
                        preferred_element_type=jnp.float32)


def _phase3_body(phi_ref, cs_ref, m2_ref, bo_ref, out_ref):
    phi = phi_ref[0]
    rs = jnp.sum(phi * cs_ref[0], axis=-1, keepdims=True) + 1e-8
    phin = phi / rs
    out_ref[0] = jnp.dot(phin.astype(jnp.bfloat16),
                         m2_ref[0].astype(jnp.bfloat16),
                         preferred_element_type=jnp.float32) + bo_ref[...]


@functools.partial(jax.jit, static_argnames=())
def kernel(query, key, value, Wq, bq, Wk, bk, Wv, bv, Wo, bo,
           splat_centers, splat_scales):
    del key, Wk, bk
    B, S, D = query.shape
    K = splat_centers.shape[0]
    NI = S // _BS

    wqt = Wq.T
    ct = splat_centers.T
    sc = splat_scales.reshape(1, K)
    bq2 = bq.reshape(1, D)
    bv2 = bv.reshape(1, D)
    bo2 = bo.reshape(1, D)

    phi, cs, p = pl.pallas_call(
        _phase1_body,
        grid=(B, NI),
        in_specs=[
            pl.BlockSpec((1, _BS, D), lambda b, i: (b, i, 0)),
            pl.BlockSpec((1, _BS, D), lambda b, i: (b, i, 0)),
            pl.BlockSpec((D, D), lambda b, i: (0, 0)),
            pl.BlockSpec((1, D), lambda b, i: (0, 0)),
            pl.BlockSpec((D, K), lambda b, i: (0, 0)),
            pl.BlockSpec((1, K), lambda b, i: (0, 0)),
        ],
        out_specs=[
            pl.BlockSpec((1, _BS, K), lambda b, i: (b, i, 0)),
            pl.BlockSpec((1, 1, K), lambda b, i: (b, 0, 0)),
            pl.BlockSpec((1, K, D), lambda b, i: (b, 0, 0)),
        ],
        out_shape=[
            jax.ShapeDtypeStruct((B, S, K), jnp.float32),
            jax.ShapeDtypeStruct((B, 1, K), jnp.float32),
            jax.ShapeDtypeStruct((B, K, D), jnp.float32),
        ],
        scratch_shapes=[pltpu.VMEM((D, D), jnp.bfloat16)],
        compiler_params=pltpu.CompilerParams(
            dimension_semantics=("parallel", "arbitrary")),
    )(query, value, wqt, bq2, ct, sc)

    m2 = pl.pallas_call(
        _phase2_body,
        grid=(B,),
        in_specs=[
            pl.BlockSpec((1, K, D), lambda b: (b, 0, 0)),
            pl.BlockSpec((1, 1, K), lambda b: (b, 0, 0)),
            pl.BlockSpec((D, D), lambda b: (0, 0)),
            pl.BlockSpec((1, D), lambda b: (0, 0)),
            pl.BlockSpec((D, D), lambda b: (0, 0)),
        ],
        out_specs=pl.BlockSpec((1, K, D), lambda b: (b, 0, 0)),
        out_shape=jax.ShapeDtypeStruct((B, K, D), jnp.float32),
        compiler_params=pltpu.CompilerParams(
            dimension_semantics=("arbitrary",)),
    )(p, cs, Wv.T, bv2, Wo.T)

    out = pl.pallas_call(
        _phase3_body,
        grid=(B, NI),
        in_specs=[
            pl.BlockSpec((1, _BS, K), lambda b, i: (b, i, 0)),
            pl.BlockSpec((1, 1, K), lambda b, i: (b, 0, 0)),
            pl.BlockSpec((1, K, D), lambda b, i: (b, 0, 0)),
            pl.BlockSpec((1, D), lambda b, i: (0, 0)),
        ],
        out_specs=pl.BlockSpec((1, _BS, D), lambda b, i: (b, i, 0)),
        out_shape=jax.ShapeDtypeStruct((B, S, D), jnp.float32),
        compiler_params=pltpu.CompilerParams(
            dimension_semantics=("parallel", "parallel")),
    )(phi, cs, m2, bo2)

    return out


# speedup vs baseline: 1.8791x; 1.0867x over previous
import functools

import jax
import jax.numpy as jnp
from jax.experimental import pallas as pl
from jax.experimental.pallas import tpu as pltpu

_BS = 1024


def _phase1_body(q_ref, v_ref, wqt_ref, bq_ref, ct_ref, sc_ref,
                 phi_ref, cs_ref, p_ref, wq_scr, ct_scr, c2_scr):
    i = pl.program_id(1)

    @pl.when(i == 0)
    def _():
        wq_scr[...] = wqt_ref[...].astype(jnp.bfloat16)
        ct = ct_ref[...]
        ct_scr[...] = ct.astype(jnp.bfloat16)
        c2_scr[...] = jnp.sum(ct * ct, axis=0, keepdims=True)

    q = jnp.dot(q_ref[0].astype(jnp.bfloat16), wq_scr[...],
                preferred_element_type=jnp.float32) + bq_ref[...]
    qb = q.astype(jnp.bfloat16)
    ones = jnp.ones((q.shape[1], 1), dtype=jnp.bfloat16)
    t2 = jnp.dot(qb * qb, ones, preferred_element_type=jnp.float32)
    qc = jnp.dot(qb, ct_scr[...], preferred_element_type=jnp.float32)
    d2 = jnp.maximum(t2 + c2_scr[...] - 2.0 * qc, 0.0)
    inv = 0.5 / (sc_ref[...] * sc_ref[...])
    phi = jnp.exp(-d2 * inv)
    phib = phi.astype(jnp.bfloat16)
    phi_ref[0] = phib
    ps = jnp.sum(phi, axis=0, keepdims=True)
    pv = jax.lax.dot_general(phib, v_ref[0].astype(jnp.bfloat16),
                             (((0,), (0,)), ((), ())),
                             preferred_element_type=jnp.float32)

    @pl.when(i == 0)
    def _():
        cs_ref[0] = ps
        p_ref[0] = pv

    @pl.when(i > 0)
    def _():
        cs_ref[0] += ps
        p_ref[0] += pv


def _phase2_body(phi_ref, cs_ref, p_ref, wvt_ref, bv_ref, wot_ref, bo_ref,
                 out_ref, m2_scr):
    i = pl.program_id(1)

    @pl.when(i == 0)
    def _():
        pv = jnp.dot(p_ref[0].astype(jnp.bfloat16),
                     wvt_ref[...].astype(jnp.bfloat16),
                     preferred_element_type=jnp.float32)
        pv = pv + jnp.transpose(cs_ref[0]) * bv_ref[...]
        m2_scr[...] = jnp.dot(pv.astype(jnp.bfloat16),
                              wot_ref[...].astype(jnp.bfloat16),
                              preferred_element_type=jnp.float32
                              ).astype(jnp.bfloat16)

    phi = phi_ref[0].astype(jnp.float32)
    rs = jnp.sum(phi * cs_ref[0], axis=-1, keepdims=True) + 1e-8
    phin = (phi / rs).astype(jnp.bfloat16)
    out_ref[0] = jnp.dot(phin, m2_scr[...],
                         preferred_element_type=jnp.float32) + bo_ref[...]


@functools.partial(jax.jit, static_argnames=())
def kernel(query, key, value, Wq, bq, Wk, bk, Wv, bv, Wo, bo,
           splat_centers, splat_scales):
    del key, Wk, bk
    B, S, D = query.shape
    K = splat_centers.shape[0]
    NI = S // _BS

    wqt = Wq.T
    ct = splat_centers.T
    sc = splat_scales.reshape(1, K)
    bq2 = bq.reshape(1, D)
    bv2 = bv.reshape(1, D)
    bo2 = bo.reshape(1, D)

    phi, cs, p = pl.pallas_call(
        _phase1_body,
        grid=(B, NI),
        in_specs=[
            pl.BlockSpec((1, _BS, D), lambda b, i: (b, i, 0)),
            pl.BlockSpec((1, _BS, D), lambda b, i: (b, i, 0)),
            pl.BlockSpec((D, D), lambda b, i: (0, 0)),
            pl.BlockSpec((1, D), lambda b, i: (0, 0)),
            pl.BlockSpec((D, K), lambda b, i: (0, 0)),
            pl.BlockSpec((1, K), lambda b, i: (0, 0)),
        ],
        out_specs=[
            pl.BlockSpec((1, _BS, K), lambda b, i: (b, i, 0)),
            pl.BlockSpec((1, 1, K), lambda b, i: (b, 0, 0)),
            pl.BlockSpec((1, K, D), lambda b, i: (b, 0, 0)),
        ],
        out_shape=[
            jax.ShapeDtypeStruct((B, S, K), jnp.bfloat16),
            jax.ShapeDtypeStruct((B, 1, K), jnp.float32),
            jax.ShapeDtypeStruct((B, K, D), jnp.float32),
        ],
        scratch_shapes=[
            pltpu.VMEM((D, D), jnp.bfloat16),
            pltpu.VMEM((D, K), jnp.bfloat16),
            pltpu.VMEM((1, K), jnp.float32),
        ],
        compiler_params=pltpu.CompilerParams(
            dimension_semantics=("parallel", "arbitrary")),
    )(query, value, wqt, bq2, ct, sc)

    out = pl.pallas_call(
        _phase2_body,
        grid=(B, NI),
        in_specs=[
            pl.BlockSpec((1, _BS, K), lambda b, i: (b, i, 0)),
            pl.BlockSpec((1, 1, K), lambda b, i: (b, 0, 0)),
            pl.BlockSpec((1, K, D), lambda b, i: (b, 0, 0)),
            pl.BlockSpec((D, D), lambda b, i: (0, 0)),
            pl.BlockSpec((1, D), lambda b, i: (0, 0)),
            pl.BlockSpec((D, D), lambda b, i: (0, 0)),
            pl.BlockSpec((1, D), lambda b, i: (0, 0)),
        ],
        out_specs=pl.BlockSpec((1, _BS, D), lambda b, i: (b, i, 0)),
        out_shape=jax.ShapeDtypeStruct((B, S, D), jnp.float32),
        scratch_shapes=[pltpu.VMEM((K, D), jnp.bfloat16)],
        compiler_params=pltpu.CompilerParams(
            dimension_semantics=("parallel", "arbitrary")),
    )(phi, cs, p, Wv.T, bv2, Wo.T, bo2)

    return out


# speedup vs baseline: 2.5519x; 1.3580x over previous
import functools

import jax
import jax.numpy as jnp
from jax.experimental import pallas as pl
from jax.experimental.pallas import tpu as pltpu

_BS = 1024


def _dot_t(a, b):
    return jax.lax.dot_general(a, b, (((1,), (1,)), ((), ())),
                               preferred_element_type=jnp.float32)


def _phase1_body(q_ref, v_ref, wq_ref, bq_ref, ct_ref, sc_ref,
                 phi_ref, cs_ref, p_ref, wq_scr, ct_scr, c2_scr):
    i = pl.program_id(1)

    @pl.when(i == 0)
    def _():
        wq_scr[...] = wq_ref[...].astype(jnp.bfloat16)
        ct = ct_ref[...]
        ct_scr[...] = ct.astype(jnp.bfloat16)
        c2_scr[...] = jnp.sum(ct * ct, axis=0, keepdims=True)

    q = _dot_t(q_ref[0].astype(jnp.bfloat16), wq_scr[...]) + bq_ref[...]
    qb = q.astype(jnp.bfloat16)
    ones = jnp.ones((q.shape[1], 1), dtype=jnp.bfloat16)
    t2 = jnp.dot(qb * qb, ones, preferred_element_type=jnp.float32)
    qc = jnp.dot(qb, ct_scr[...], preferred_element_type=jnp.float32)
    d2 = jnp.maximum(t2 + c2_scr[...] - 2.0 * qc, 0.0)
    inv = 0.5 / (sc_ref[...] * sc_ref[...])
    phi = jnp.exp(-d2 * inv)
    phib = phi.astype(jnp.bfloat16)
    phi_ref[0] = phib
    ps = jnp.sum(phi, axis=0, keepdims=True)
    pv = jax.lax.dot_general(phib, v_ref[0].astype(jnp.bfloat16),
                             (((0,), (0,)), ((), ())),
                             preferred_element_type=jnp.float32)

    @pl.when(i == 0)
    def _():
        cs_ref[0] = ps
        p_ref[0] = pv

    @pl.when(i > 0)
    def _():
        cs_ref[0] += ps
        p_ref[0] += pv


def _phase2_body(phi_ref, cs_ref, p_ref, wv_ref, bv_ref, wo_ref, bo_ref,
                 out_ref, m2_scr):
    i = pl.program_id(1)

    @pl.when(i == 0)
    def _():
        pv = _dot_t(p_ref[0].astype(jnp.bfloat16),
                    wv_ref[...].astype(jnp.bfloat16))
        pv = pv + jnp.transpose(cs_ref[0]) * bv_ref[...]
        m2_scr[...] = _dot_t(pv.astype(jnp.bfloat16),
                             wo_ref[...].astype(jnp.bfloat16)
                             ).astype(jnp.bfloat16)

    phi = phi_ref[0].astype(jnp.float32)
    rs = jnp.sum(phi * cs_ref[0], axis=-1, keepdims=True) + 1e-8
    phin = (phi / rs).astype(jnp.bfloat16)
    out_ref[0] = jnp.dot(phin, m2_scr[...],
                         preferred_element_type=jnp.float32) + bo_ref[...]


@functools.partial(jax.jit, static_argnames=())
def kernel(query, key, value, Wq, bq, Wk, bk, Wv, bv, Wo, bo,
           splat_centers, splat_scales):
    del key, Wk, bk
    B, S, D = query.shape
    K = splat_centers.shape[0]
    NI = S // _BS

    ct = splat_centers.T
    sc = splat_scales.reshape(1, K)
    bq2 = bq.reshape(1, D)
    bv2 = bv.reshape(1, D)
    bo2 = bo.reshape(1, D)

    phi, cs, p = pl.pallas_call(
        _phase1_body,
        grid=(B, NI),
        in_specs=[
            pl.BlockSpec((1, _BS, D), lambda b, i: (b, i, 0)),
            pl.BlockSpec((1, _BS, D), lambda b, i: (b, i, 0)),
            pl.BlockSpec((D, D), lambda b, i: (0, 0)),
            pl.BlockSpec((1, D), lambda b, i: (0, 0)),
            pl.BlockSpec((D, K), lambda b, i: (0, 0)),
            pl.BlockSpec((1, K), lambda b, i: (0, 0)),
        ],
        out_specs=[
            pl.BlockSpec((1, _BS, K), lambda b, i: (b, i, 0)),
            pl.BlockSpec((1, 1, K), lambda b, i: (b, 0, 0)),
            pl.BlockSpec((1, K, D), lambda b, i: (b, 0, 0)),
        ],
        out_shape=[
            jax.ShapeDtypeStruct((B, S, K), jnp.bfloat16),
            jax.ShapeDtypeStruct((B, 1, K), jnp.float32),
            jax.ShapeDtypeStruct((B, K, D), jnp.float32),
        ],
        scratch_shapes=[
            pltpu.VMEM((D, D), jnp.bfloat16),
            pltpu.VMEM((D, K), jnp.bfloat16),
            pltpu.VMEM((1, K), jnp.float32),
        ],
        compiler_params=pltpu.CompilerParams(
            dimension_semantics=("parallel", "arbitrary")),
    )(query, value, Wq, bq2, ct, sc)

    out = pl.pallas_call(
        _phase2_body,
        grid=(B, NI),
        in_specs=[
            pl.BlockSpec((1, _BS, K), lambda b, i: (b, i, 0)),
            pl.BlockSpec((1, 1, K), lambda b, i: (b, 0, 0)),
            pl.BlockSpec((1, K, D), lambda b, i: (b, 0, 0)),
            pl.BlockSpec((D, D), lambda b, i: (0, 0)),
            pl.BlockSpec((1, D), lambda b, i: (0, 0)),
            pl.BlockSpec((D, D), lambda b, i: (0, 0)),
            pl.BlockSpec((1, D), lambda b, i: (0, 0)),
        ],
        out_specs=pl.BlockSpec((1, _BS, D), lambda b, i: (b, i, 0)),
        out_shape=jax.ShapeDtypeStruct((B, S, D), jnp.float32),
        scratch_shapes=[pltpu.VMEM((K, D), jnp.bfloat16)],
        compiler_params=pltpu.CompilerParams(
            dimension_semantics=("parallel", "arbitrary")),
    )(phi, cs, p, Wv, bv2, Wo, bo2)

    return out
